# Initial kernel scaffold; baseline (speedup 1.0000x reference)
import jax
import jax.numpy as jnp
from jax.experimental import pallas as pl


def kernel(input):
    raise NotImplementedError("write your pallas kernel here")


# speedup vs baseline: 3.9642x; 3.9642x over previous
import functools

import jax
import jax.numpy as jnp
from jax import lax
from jax.experimental import pallas as pl
from jax.experimental.pallas import tpu as pltpu
from jax.experimental.pallas import tpu_sc as plsc

B, C, H, W = 8, 96, 112, 112
N = B * C * H
W_OUT = 4 * W
NW = 32
ROWS_PER_W = N // NW
CHUNK = 64
NCHUNKS = ROWS_PER_W // CHUNK
L = 16


def _sc_unpool(x_flat):
    mesh = plsc.VectorSubcoreMesh(core_axis_name="c", subcore_axis_name="s")

    @functools.partial(
        pl.kernel,
        mesh=mesh,
        out_type=jax.ShapeDtypeStruct((N * W_OUT,), jnp.float32),
        compiler_params=pltpu.CompilerParams(needs_layout_passes=False),
        scratch_types=[
            pltpu.VMEM((CHUNK * W,), jnp.float32),
            pltpu.VMEM((CHUNK * W_OUT,), jnp.float32),
        ],
    )
    def k(in_hbm, out_hbm, in_v, out_v):
        wid = lax.axis_index("s") * 2 + lax.axis_index("c")
        base_row = wid * ROWS_PER_W
        iota = lax.iota(jnp.int32, L)
        zeros = jnp.zeros((L,), jnp.float32)

        def zbody(i, _):
            out_v[pl.ds(i * L, L)] = zeros
            return 0

        lax.fori_loop(0, CHUNK * W_OUT // L, zbody, 0)

        def cbody(ci, _):
            row0 = base_row + ci * CHUNK
            pltpu.sync_copy(in_hbm.at[pl.ds(row0 * W, CHUNK * W)], in_v)

            def rbody(r, _):
                row_out = out_v.at[pl.ds(r * W_OUT, W_OUT)]
                for g in range(W // L):
                    vals = in_v[pl.ds(r * W + g * L, L)]
                    idx = (2 * W + 1 + 2 * L * g) + 2 * iota
                    plsc.store_scatter(row_out, [idx], vals)
                return 0

            lax.fori_loop(0, CHUNK, rbody, 0)
            pltpu.sync_copy(out_v, out_hbm.at[pl.ds(row0 * W_OUT, CHUNK * W_OUT)])
            return 0

        lax.fori_loop(0, NCHUNKS, cbody, 0)

    return k(x_flat)


def kernel(input):
    x_flat = input.reshape(N * W)
    out_flat = _sc_unpool(x_flat)
    return out_flat.reshape(B, C, 2 * H, 2 * W)


# speedup vs baseline: 5.0379x; 1.2709x over previous
import functools

import jax
import jax.numpy as jnp
from jax import lax
from jax.experimental import pallas as pl
from jax.experimental.pallas import tpu as pltpu
from jax.experimental.pallas import tpu_sc as plsc

B, C, H, W = 8, 96, 112, 112
N = B * C * H
W_OUT = 4 * W
NW = 32
ROWS_PER_W = N // NW
CHUNK = 64
NCHUNKS = ROWS_PER_W // CHUNK
L = 16


def _sc_unpool(x_flat):
    mesh = plsc.VectorSubcoreMesh(core_axis_name="c", subcore_axis_name="s")

    @functools.partial(
        pl.kernel,
        mesh=mesh,
        out_type=jax.ShapeDtypeStruct((N * W_OUT,), jnp.float32),
        compiler_params=pltpu.CompilerParams(needs_layout_passes=False),
        scratch_types=[
            pltpu.VMEM((CHUNK * W,), jnp.float32),
            pltpu.VMEM((CHUNK * W,), jnp.float32),
            pltpu.VMEM((CHUNK * W_OUT,), jnp.float32),
            pltpu.VMEM((CHUNK * W_OUT,), jnp.float32),
            pltpu.SemaphoreType.DMA,
            pltpu.SemaphoreType.DMA,
            pltpu.SemaphoreType.DMA,
            pltpu.SemaphoreType.DMA,
        ],
    )
    def k(in_hbm, out_hbm, iv0, iv1, ov0, ov1, si0, si1, so0, so1):
        in_bufs = (iv0, iv1)
        out_bufs = (ov0, ov1)
        wid = lax.axis_index("s") * 2 + lax.axis_index("c")
        base_row = wid * ROWS_PER_W
        in_sems = (si0, si1)
        out_sems = (so0, so1)
        iota = lax.iota(jnp.int32, L)
        zeros = jnp.zeros((L,), jnp.float32)

        def in_slice(ci):
            return in_hbm.at[pl.ds((base_row + ci * CHUNK) * W, CHUNK * W)]

        def out_slice(ci):
            return out_hbm.at[
                pl.ds((base_row + ci * CHUNK) * W_OUT, CHUNK * W_OUT)
            ]

        def zbody(i, _):
            for bb in range(2):
                out_bufs[bb][pl.ds(i * L, L)] = zeros
            return 0

        lax.fori_loop(0, CHUNK * W_OUT // L, zbody, 0)

        pltpu.async_copy(in_slice(0), in_bufs[0], in_sems[0])

        def step(i, _):
            for b in range(2):
                ci = i * 2 + b
                nxt = ci + 1

                @pl.when(nxt < NCHUNKS)
                def _():
                    pltpu.async_copy(
                        in_slice(nxt), in_bufs[1 - b], in_sems[1 - b]
                    )

                pltpu.make_async_copy(
                    in_slice(ci), in_bufs[b], in_sems[b]
                ).wait()

                @pl.when(ci >= 2)
                def _():
                    pltpu.make_async_copy(
                        out_bufs[b], out_slice(ci), out_sems[b]
                    ).wait()

                def rbody(r, _):
                    row_out = out_bufs[b].at[pl.ds(r * W_OUT, W_OUT)]
                    for g in range(W // L):
                        vals = in_bufs[b][pl.ds(r * W + g * L, L)]
                        idx = (2 * W + 1 + 2 * L * g) + 2 * iota
                        plsc.store_scatter(row_out, [idx], vals)
                    return 0

                lax.fori_loop(0, CHUNK, rbody, 0)
                pltpu.async_copy(out_bufs[b], out_slice(ci), out_sems[b])
            return 0

        lax.fori_loop(0, NCHUNKS // 2, step, 0)
        pltpu.make_async_copy(
            out_bufs[0], out_slice(NCHUNKS - 2), out_sems[0]
        ).wait()
        pltpu.make_async_copy(
            out_bufs[1], out_slice(NCHUNKS - 1), out_sems[1]
        ).wait()

    return k(x_flat)


def kernel(input):
    x_flat = input.reshape(N * W)
    out_flat = _sc_unpool(x_flat)
    return out_flat.reshape(B, C, 2 * H, 2 * W)


# speedup vs baseline: 9.4587x; 1.8775x over previous
import functools

import jax
import jax.numpy as jnp
from jax import lax
from jax.experimental import pallas as pl
from jax.experimental.pallas import tpu as pltpu
from jax.experimental.pallas import tpu_sc as plsc

B, C, H, W = 8, 96, 112, 112
NIMG = B * C
NW = 32
IMGS_PER_W = NIMG // NW
L = 16


def _sc_unpool(x):
    mesh = plsc.VectorSubcoreMesh(core_axis_name="c", subcore_axis_name="s")

    @functools.partial(
        pl.kernel,
        mesh=mesh,
        out_type=jax.ShapeDtypeStruct((NIMG, 2 * H, 2 * W), jnp.float32),
        compiler_params=pltpu.CompilerParams(needs_layout_passes=False),
        scratch_types=[
            pltpu.VMEM((1, H, W), jnp.float32),
            pltpu.VMEM((1, 2 * H, 2 * W), jnp.float32),
        ],
    )
    def k(in_hbm, out_hbm, in_v, out_v):
        wid = lax.axis_index("s") * 2 + lax.axis_index("c")
        img0 = wid * IMGS_PER_W
        iota = lax.iota(jnp.int32, L)
        zeros = jnp.zeros((L,), jnp.float32)

        def zrow(r, _):
            for g in range(2 * W // L):
                out_v[0, r, pl.ds(g * L, L)] = zeros
            return 0

        lax.fori_loop(0, 2 * H, zrow, 0)

        def ibody(t, _):
            img = img0 + t
            pltpu.sync_copy(in_hbm.at[pl.ds(img, 1)], in_v)

            def rbody(r, _):
                zi = iota * 0
                row_idx = zi + (2 * r + 1)
                for g in range(W // L):
                    vals = in_v[0, r, pl.ds(g * L, L)]
                    idx = (2 * L * g + 1) + 2 * iota
                    plsc.store_scatter(out_v, [zi, row_idx, idx], vals)
                return 0

            lax.fori_loop(0, H, rbody, 0)
            pltpu.sync_copy(out_v, out_hbm.at[pl.ds(img, 1)])
            return 0

        lax.fori_loop(0, IMGS_PER_W, ibody, 0)

    return k(x)


def kernel(input):
    out = _sc_unpool(input.reshape(NIMG, H, W))
    return out.reshape(B, C, 2 * H, 2 * W)


# speedup vs baseline: 15.6680x; 1.6565x over previous
import functools

import jax
import jax.numpy as jnp
from jax import lax
from jax.experimental import pallas as pl
from jax.experimental.pallas import tpu as pltpu
from jax.experimental.pallas import tpu_sc as plsc

B, C, H, W = 8, 96, 112, 112
NIMG = B * C
NW = 32
IMGS_PER_W = NIMG // NW
HH = H // 2
UNITS_PER_W = IMGS_PER_W * 2
L = 16


def _sc_unpool(x):
    mesh = plsc.VectorSubcoreMesh(core_axis_name="c", subcore_axis_name="s")

    @functools.partial(
        pl.kernel,
        mesh=mesh,
        out_type=jax.ShapeDtypeStruct((NIMG, 2 * H, 2 * W), jnp.float32),
        compiler_params=pltpu.CompilerParams(needs_layout_passes=False),
        scratch_types=[
            pltpu.VMEM((1, HH, W), jnp.float32),
            pltpu.VMEM((1, HH, W), jnp.float32),
            pltpu.VMEM((1, 2 * HH, 2 * W), jnp.float32),
            pltpu.VMEM((1, 2 * HH, 2 * W), jnp.float32),
            pltpu.SemaphoreType.DMA,
            pltpu.SemaphoreType.DMA,
            pltpu.SemaphoreType.DMA,
            pltpu.SemaphoreType.DMA,
        ],
    )
    def k(in_hbm, out_hbm, iv0, iv1, ov0, ov1, si0, si1, so0, so1):
        in_bufs = (iv0, iv1)
        out_bufs = (ov0, ov1)
        in_sems = (si0, si1)
        out_sems = (so0, so1)
        wid = lax.axis_index("s") * 2 + lax.axis_index("c")
        img0 = wid * IMGS_PER_W
        iota = lax.iota(jnp.int32, L)
        zeros = jnp.zeros((L,), jnp.float32)

        def in_slice(u):
            return in_hbm.at[
                pl.ds(img0 + u // 2, 1), pl.ds((u % 2) * HH, HH)
            ]

        def out_slice(u):
            return out_hbm.at[
                pl.ds(img0 + u // 2, 1), pl.ds((u % 2) * 2 * HH, 2 * HH)
            ]

        def zrow(r, _):
            for bb in range(2):
                for g in range(2 * W // L):
                    out_bufs[bb][0, r, pl.ds(g * L, L)] = zeros
            return 0

        lax.fori_loop(0, 2 * HH, zrow, 0)

        pltpu.async_copy(in_slice(0), in_bufs[0], in_sems[0])

        def step(i, _):
            for b in range(2):
                u = i * 2 + b
                nxt = u + 1

                @pl.when(nxt < UNITS_PER_W)
                def _():
                    pltpu.async_copy(
                        in_slice(nxt), in_bufs[1 - b], in_sems[1 - b]
                    )

                pltpu.make_async_copy(
                    in_slice(u), in_bufs[b], in_sems[b]
                ).wait()

                @pl.when(u >= 2)
                def _():
                    pltpu.make_async_copy(
                        out_bufs[b], out_slice(u), out_sems[b]
                    ).wait()

                def rbody(r, _):
                    zi = iota * 0
                    row_idx = zi + (2 * r + 1)
                    for g in range(W // L):
                        vals = in_bufs[b][0, r, pl.ds(g * L, L)]
                        idx = (2 * L * g + 1) + 2 * iota
                        plsc.store_scatter(
                            out_bufs[b], [zi, row_idx, idx], vals
                        )
                    return 0

                lax.fori_loop(0, HH, rbody, 0)
                pltpu.async_copy(out_bufs[b], out_slice(u), out_sems[b])
            return 0

        lax.fori_loop(0, UNITS_PER_W // 2, step, 0)
        pltpu.make_async_copy(
            out_bufs[0], out_slice(UNITS_PER_W - 2), out_sems[0]
        ).wait()
        pltpu.make_async_copy(
            out_bufs[1], out_slice(UNITS_PER_W - 1), out_sems[1]
        ).wait()

    return k(x)


def kernel(input):
    out = _sc_unpool(input.reshape(NIMG, H, W))
    return out.reshape(B, C, 2 * H, 2 * W)
